# baseline (speedup 1.0000x reference)
import functools

import jax
import jax.numpy as jnp
from jax import lax
from jax.experimental import pallas as pl
from jax.experimental.pallas import tpu as pltpu
from jax.experimental.pallas import tpu_sc as plsc


def _route_sc(table, tid, n_tokens, vocab, ei):
    info = plsc.get_sparse_core_info()
    nc, ns, nl = info.num_cores, info.num_subcores, info.num_lanes
    nw = nc * ns
    bpw = n_tokens // nw
    assert n_tokens % nw == 0 and bpw % 8 == 0 and bpw % nl == 0

    mesh = plsc.VectorSubcoreMesh(core_axis_name="c", subcore_axis_name="s")

    @functools.partial(
        pl.kernel,
        mesh=mesh,
        out_type=jax.ShapeDtypeStruct((n_tokens,), jnp.int32),
        scratch_types=[
            pltpu.VMEM((bpw,), jnp.int32),
            pltpu.VMEM((bpw,), jnp.int32),
            pltpu.SemaphoreType.DMA,
        ],
    )
    def route(table_hbm, tid_hbm, out_hbm, tid_v, eid_v, sem):
        wid = lax.axis_index("s") * nc + lax.axis_index("c")
        base = wid * bpw
        pltpu.sync_copy(tid_hbm.at[pl.ds(base, bpw)], tid_v)
        for i in range(bpw // nl):
            sl = pl.ds(i * nl, nl)
            v = tid_v[sl]
            tid_v[sl] = jnp.minimum(jnp.maximum(v, 0), vocab - 1)
        pltpu.async_copy(table_hbm.at[tid_v], eid_v, sem).wait()
        for i in range(bpw // nl):
            sl = pl.ds(i * nl, nl)
            eid_v[sl] = eid_v[sl] * ei
        pltpu.sync_copy(eid_v, out_hbm.at[pl.ds(base, bpw)])

    return route(table, tid)


def _mlp_body(x_ref, wg_hbm, wu_hbm, wd_ref, cb_ref, o_ref, wg_cat, wu_cat,
              stage, sem, *, ei, ne):
    i = pl.program_id(0)
    gsz = stage.shape[0]

    @pl.when(i == 0)
    def _():
        for src, cat in ((wg_hbm, wg_cat), (wu_hbm, wu_cat)):
            for g0 in range(0, ne, gsz):
                c = pltpu.make_async_copy(src.at[pl.ds(g0, gsz)], stage, sem)
                c.start()
                c.wait()
                for k in range(gsz):
                    cat[:, pl.ds((g0 + k) * ei, ei)] = stage[k]

    x = x_ref[:]
    g = jnp.dot(x, wg_cat[:], preferred_element_type=jnp.float32)
    u = jnp.dot(x, wu_cat[:], preferred_element_type=jnp.float32)
    col = lax.broadcasted_iota(jnp.int32, g.shape, 1)
    base = cb_ref[:]
    mask = (col >= base) & (col < base + ei)
    act = jnp.where(mask, g * lax.logistic(g) * u, 0.0)
    o_ref[:] = jnp.dot(act, wd_ref[:], preferred_element_type=jnp.float32)


def kernel(hidden_states, token_ids, gate_proj, up_proj, down_proj, token_to_expert):
    b, s, h = hidden_states.shape
    e, _, ei = gate_proj.shape
    inter = e * ei
    vocab = token_to_expert.shape[0]
    n = b * s

    x = hidden_states.reshape(n, h)
    tid = token_ids.reshape(n)

    col_base = _route_sc(token_to_expert, tid, n, vocab, ei).reshape(n, 1)

    wd = down_proj.reshape(inter, h)

    tn = 256
    while n % tn:
        tn //= 2
    grid = (n // tn,)

    out = pl.pallas_call(
        functools.partial(_mlp_body, ei=ei, ne=e),
        grid=grid,
        in_specs=[
            pl.BlockSpec((tn, h), lambda i: (i, 0)),
            pl.BlockSpec(memory_space=pl.ANY),
            pl.BlockSpec(memory_space=pl.ANY),
            pl.BlockSpec((inter, h), lambda i: (0, 0)),
            pl.BlockSpec((tn, 1), lambda i: (i, 0)),
        ],
        out_specs=pl.BlockSpec((tn, h), lambda i: (i, 0)),
        out_shape=jax.ShapeDtypeStruct((n, h), jnp.float32),
        scratch_shapes=[
            pltpu.VMEM((h, inter), jnp.float32),
            pltpu.VMEM((h, inter), jnp.float32),
            pltpu.VMEM((8, h, ei), jnp.float32),
            pltpu.SemaphoreType.DMA,
        ],
    )(x, gate_proj, up_proj, wd, col_base)

    return out.reshape(b, s, h)


# speedup vs baseline: 1.6394x; 1.6394x over previous
import functools

import jax
import jax.numpy as jnp
from jax import lax
from jax.experimental import pallas as pl
from jax.experimental.pallas import tpu as pltpu
from jax.experimental.pallas import tpu_sc as plsc


def _route_sc(table, tid, n_tokens, vocab, ei):
    info = plsc.get_sparse_core_info()
    nc, ns, nl = info.num_cores, info.num_subcores, info.num_lanes
    nw = nc * ns
    bpw = n_tokens // nw
    assert n_tokens % nw == 0 and bpw % 8 == 0 and bpw % nl == 0

    mesh = plsc.VectorSubcoreMesh(core_axis_name="c", subcore_axis_name="s")

    @functools.partial(
        pl.kernel,
        mesh=mesh,
        out_type=jax.ShapeDtypeStruct((n_tokens,), jnp.int32),
        scratch_types=[
            pltpu.VMEM((bpw,), jnp.int32),
            pltpu.VMEM((bpw,), jnp.int32),
            pltpu.SemaphoreType.DMA,
        ],
    )
    def route(table_hbm, tid_hbm, out_hbm, tid_v, eid_v, sem):
        wid = lax.axis_index("s") * nc + lax.axis_index("c")
        base = wid * bpw
        pltpu.sync_copy(tid_hbm.at[pl.ds(base, bpw)], tid_v)
        for i in range(bpw // nl):
            sl = pl.ds(i * nl, nl)
            v = tid_v[sl]
            tid_v[sl] = jnp.minimum(jnp.maximum(v, 0), vocab - 1)
        pltpu.async_copy(table_hbm.at[tid_v], eid_v, sem).wait()
        for i in range(bpw // nl):
            sl = pl.ds(i * nl, nl)
            eid_v[sl] = eid_v[sl] * ei
        pltpu.sync_copy(eid_v, out_hbm.at[pl.ds(base, bpw)])

    return route(table, tid)


def _mlp_body(x_ref, wg_ref, wu_ref, wd_ref, cb_ref, o_ref, *, ei):
    x = x_ref[:]
    g = jnp.dot(x, wg_ref[:], preferred_element_type=jnp.float32)
    u = jnp.dot(x, wu_ref[:], preferred_element_type=jnp.float32)
    col = lax.broadcasted_iota(jnp.int32, g.shape, 1)
    base = cb_ref[:]
    mask = (col >= base) & (col < base + ei)
    act = jnp.where(mask, g * lax.logistic(g) * u, 0.0)
    o_ref[:] = jnp.dot(act, wd_ref[:], preferred_element_type=jnp.float32)


def kernel(hidden_states, token_ids, gate_proj, up_proj, down_proj, token_to_expert):
    b, s, h = hidden_states.shape
    e, _, ei = gate_proj.shape
    inter = e * ei
    vocab = token_to_expert.shape[0]
    n = b * s

    x = hidden_states.reshape(n, h)
    tid = token_ids.reshape(n)

    col_base = _route_sc(token_to_expert, tid, n, vocab, ei).reshape(n, 1)

    wg = gate_proj.transpose(1, 0, 2).reshape(h, inter)
    wu = up_proj.transpose(1, 0, 2).reshape(h, inter)
    wd = down_proj.reshape(inter, h)

    tn = 512
    while n % tn:
        tn //= 2
    grid = (n // tn,)

    out = pl.pallas_call(
        functools.partial(_mlp_body, ei=ei),
        grid=grid,
        in_specs=[
            pl.BlockSpec((tn, h), lambda i: (i, 0)),
            pl.BlockSpec((h, inter), lambda i: (0, 0)),
            pl.BlockSpec((h, inter), lambda i: (0, 0)),
            pl.BlockSpec((inter, h), lambda i: (0, 0)),
            pl.BlockSpec((tn, 1), lambda i: (i, 0)),
        ],
        out_specs=pl.BlockSpec((tn, h), lambda i: (i, 0)),
        out_shape=jax.ShapeDtypeStruct((n, h), jnp.float32),
    )(x, wg, wu, wd, col_base)

    return out.reshape(b, s, h)


# speedup vs baseline: 1.6492x; 1.0060x over previous
import functools

import jax
import jax.numpy as jnp
from jax import lax
from jax.experimental import pallas as pl
from jax.experimental.pallas import tpu as pltpu
from jax.experimental.pallas import tpu_sc as plsc


def _route_sc(table, tid, n_tokens, vocab, ei):
    info = plsc.get_sparse_core_info()
    nc, ns, nl = info.num_cores, info.num_subcores, info.num_lanes
    nw = nc * ns
    bpw = n_tokens // nw
    assert n_tokens % nw == 0 and bpw % 8 == 0 and bpw % nl == 0

    mesh = plsc.VectorSubcoreMesh(core_axis_name="c", subcore_axis_name="s")

    @functools.partial(
        pl.kernel,
        mesh=mesh,
        out_type=jax.ShapeDtypeStruct((n_tokens,), jnp.int32),
        scratch_types=[
            pltpu.VMEM((bpw,), jnp.int32),
            pltpu.VMEM((bpw,), jnp.int32),
            pltpu.SemaphoreType.DMA,
        ],
    )
    def route(table_hbm, tid_hbm, out_hbm, tid_v, eid_v, sem):
        wid = lax.axis_index("s") * nc + lax.axis_index("c")
        base = wid * bpw
        pltpu.sync_copy(tid_hbm.at[pl.ds(base, bpw)], tid_v)
        for i in range(bpw // nl):
            sl = pl.ds(i * nl, nl)
            v = tid_v[sl]
            tid_v[sl] = jnp.minimum(jnp.maximum(v, 0), vocab - 1)
        pltpu.async_copy(table_hbm.at[tid_v], eid_v, sem).wait()
        for i in range(bpw // nl):
            sl = pl.ds(i * nl, nl)
            eid_v[sl] = eid_v[sl] * ei
        pltpu.sync_copy(eid_v, out_hbm.at[pl.ds(base, bpw)])

    return route(table, tid)


def _mlp_body(x_ref, wg_ref, wu_ref, wd_ref, cb_ref, o_ref, *, ei):
    x = x_ref[:]
    g = jnp.dot(x, wg_ref[:], preferred_element_type=jnp.float32)
    u = jnp.dot(x, wu_ref[:], preferred_element_type=jnp.float32)
    col = lax.broadcasted_iota(jnp.int32, g.shape, 1)
    base = cb_ref[:]
    mask = (col - base).astype(jnp.uint32) < jnp.uint32(ei)
    act = jnp.where(mask, g * lax.logistic(g) * u, 0.0)
    o_ref[:] = jnp.dot(act, wd_ref[:], preferred_element_type=jnp.float32)


def kernel(hidden_states, token_ids, gate_proj, up_proj, down_proj, token_to_expert):
    b, s, h = hidden_states.shape
    e, _, ei = gate_proj.shape
    inter = e * ei
    vocab = token_to_expert.shape[0]
    n = b * s

    x = hidden_states.reshape(n, h)
    tid = token_ids.reshape(n)

    col_base = _route_sc(token_to_expert, tid, n, vocab, ei).reshape(n, 1)

    wg = gate_proj.transpose(1, 0, 2).reshape(h, inter)
    wu = up_proj.transpose(1, 0, 2).reshape(h, inter)
    wd = down_proj.reshape(inter, h)

    tn = 512
    while n % tn:
        tn //= 2
    grid = (n // tn,)

    out = pl.pallas_call(
        functools.partial(_mlp_body, ei=ei),
        grid=grid,
        in_specs=[
            pl.BlockSpec((tn, h), lambda i: (i, 0)),
            pl.BlockSpec((h, inter), lambda i: (0, 0)),
            pl.BlockSpec((h, inter), lambda i: (0, 0)),
            pl.BlockSpec((inter, h), lambda i: (0, 0)),
            pl.BlockSpec((tn, 1), lambda i: (i, 0)),
        ],
        out_specs=pl.BlockSpec((tn, h), lambda i: (i, 0)),
        out_shape=jax.ShapeDtypeStruct((n, h), jnp.float32),
    )(x, wg, wu, wd, col_base)

    return out.reshape(b, s, h)
